# Initial kernel scaffold; baseline (speedup 1.0000x reference)
import jax
import jax.numpy as jnp
from jax.experimental import pallas as pl


def kernel(x, coords, pos_table):
    raise NotImplementedError("write your pallas kernel here")


# speedup vs baseline: 1.9503x; 1.9503x over previous
import functools

import jax
import jax.numpy as jnp
from jax import lax
from jax.experimental import pallas as pl
from jax.experimental.pallas import tpu as pltpu, tpu_sc as plsc

D_MODEL = 256
HALF = 128
TABLE_ROWS = 512
BATCH = 16384

_info = plsc.get_sparse_core_info()
NUM_CORES = _info.num_cores
NUM_SUBCORES = _info.num_subcores
NUM_WORKERS = NUM_CORES * NUM_SUBCORES
ROWS_PER_WORKER = BATCH // NUM_WORKERS
CHUNK = 128
CHUNKS_PER_WORKER = ROWS_PER_WORKER // CHUNK
LANES = 16


def _sc_body(x_hbm, widx_hbm, hidx_hbm, wt_hbm, ht_hbm, out_hbm,
             widx_v, hidx_v, xb, wr, hr, sem_w, sem_h):
    wid = lax.axis_index("s") * NUM_CORES + lax.axis_index("c")
    base = wid * ROWS_PER_WORKER
    irow = wid * CHUNKS_PER_WORKER

    pltpu.sync_copy(widx_hbm.at[pl.ds(irow, CHUNKS_PER_WORKER)], widx_v)
    pltpu.sync_copy(hidx_hbm.at[pl.ds(irow, CHUNKS_PER_WORKER)], hidx_v)

    for j in range(CHUNKS_PER_WORKER):
        row0 = base + j * CHUNK
        cw = pltpu.async_copy(wt_hbm.at[widx_v.at[j]], wr, sem_w)
        ch = pltpu.async_copy(ht_hbm.at[hidx_v.at[j]], hr, sem_h)
        pltpu.sync_copy(x_hbm.at[pl.ds(row0, CHUNK)], xb)
        cw.wait()
        ch.wait()

        def row_body(r, _):
            for t in range(HALF // LANES):
                o = t * LANES
                xb[r, pl.ds(o, LANES)] = (
                    xb[r, pl.ds(o, LANES)] + wr[r, pl.ds(o, LANES)])
                xb[r, pl.ds(HALF + o, LANES)] = (
                    xb[r, pl.ds(HALF + o, LANES)] + hr[r, pl.ds(o, LANES)])
            return 0

        lax.fori_loop(0, CHUNK, row_body, 0)
        pltpu.sync_copy(xb, out_hbm.at[pl.ds(row0, CHUNK)])


@jax.jit
def _pos_encode_add(x, coords, pos_table):
    wt = jnp.transpose(pos_table[:HALF, 0, :])
    ht = jnp.transpose(pos_table[HALF:, :, 0])
    widx = coords[:, 3].reshape(BATCH // CHUNK, CHUNK)
    hidx = coords[:, 2].reshape(BATCH // CHUNK, CHUNK)

    mesh = plsc.VectorSubcoreMesh(core_axis_name="c", subcore_axis_name="s")
    run = pl.kernel(
        _sc_body,
        out_type=jax.ShapeDtypeStruct((BATCH, D_MODEL), jnp.float32),
        mesh=mesh,
        scratch_types=[
            pltpu.VMEM((CHUNKS_PER_WORKER, CHUNK), jnp.int32),
            pltpu.VMEM((CHUNKS_PER_WORKER, CHUNK), jnp.int32),
            pltpu.VMEM((CHUNK, D_MODEL), jnp.float32),
            pltpu.VMEM((CHUNK, HALF), jnp.float32),
            pltpu.VMEM((CHUNK, HALF), jnp.float32),
            pltpu.SemaphoreType.DMA,
            pltpu.SemaphoreType.DMA,
        ],
    )
    return run(x, widx, hidx, wt, ht)


def kernel(x, coords, pos_table):
    return _pos_encode_add(x, coords, pos_table)


# speedup vs baseline: 2.6694x; 1.3687x over previous
import jax
import jax.numpy as jnp
from jax import lax
from jax.experimental import pallas as pl
from jax.experimental.pallas import tpu as pltpu, tpu_sc as plsc

D_MODEL = 256
HALF = 128
TABLE_ROWS = 512
BATCH = 16384

_info = plsc.get_sparse_core_info()
NUM_CORES = _info.num_cores
NUM_SUBCORES = _info.num_subcores
NUM_WORKERS = NUM_CORES * NUM_SUBCORES
ROWS_PER_WORKER = BATCH // NUM_WORKERS
CHUNK = 64
CHUNKS_PER_WORKER = ROWS_PER_WORKER // CHUNK
LANES = 16


def _sc_body(x_hbm, widx_hbm, hidx_hbm, tab_hbm, out_hbm,
             widx_v, hidx_v, xb, wr, hr,
             sx0, sx1, sw0, sw1, sh0, sh1, so0, so1):
    wid = lax.axis_index("s") * NUM_CORES + lax.axis_index("c")
    base = wid * ROWS_PER_WORKER
    irow = wid * CHUNKS_PER_WORKER

    pltpu.sync_copy(widx_hbm.at[pl.ds(irow, CHUNKS_PER_WORKER)], widx_v)
    pltpu.sync_copy(hidx_hbm.at[pl.ds(irow, CHUNKS_PER_WORKER)], hidx_v)

    sx = (sx0, sx1)
    sw = (sw0, sw1)
    sh = (sh0, sh1)
    so = (so0, so1)

    def issue(j):
        b = j & 1
        return (
            pltpu.async_copy(
                x_hbm.at[pl.ds(base + j * CHUNK, CHUNK)], xb.at[b], sx[b]),
            pltpu.async_copy(tab_hbm.at[widx_v.at[j]], wr.at[b], sw[b]),
            pltpu.async_copy(tab_hbm.at[hidx_v.at[j]], hr.at[b], sh[b]),
        )

    descs = [None] * CHUNKS_PER_WORKER
    outd = [None, None]
    descs[0] = issue(0)
    for j in range(CHUNKS_PER_WORKER):
        b = j & 1
        if j + 1 < CHUNKS_PER_WORKER:
            if outd[1 - b] is not None:
                outd[1 - b].wait()
            descs[j + 1] = issue(j + 1)
        for d in descs[j]:
            d.wait()

        @plsc.parallel_loop(0, CHUNK, unroll=4)
        def row_body(r):
            for t in range(HALF // LANES):
                o = t * LANES
                xb[b, r, pl.ds(o, LANES)] = (
                    xb[b, r, pl.ds(o, LANES)] + wr[b, r, pl.ds(o, LANES)])
                xb[b, r, pl.ds(HALF + o, LANES)] = (
                    xb[b, r, pl.ds(HALF + o, LANES)]
                    + hr[b, r, pl.ds(o, LANES)])

        outd[b] = pltpu.async_copy(
            xb.at[b], out_hbm.at[pl.ds(base + j * CHUNK, CHUNK)], so[b])
    outd[0].wait()
    outd[1].wait()


@jax.jit
def _pos_encode_add(x, coords, pos_table):
    tab = jnp.transpose(jnp.concatenate(
        [pos_table[:HALF, 0, :], pos_table[HALF:, :, 0]], axis=1))
    widx = coords[:, 3].reshape(BATCH // CHUNK, CHUNK)
    hidx = (coords[:, 2] + TABLE_ROWS).reshape(BATCH // CHUNK, CHUNK)

    mesh = plsc.VectorSubcoreMesh(core_axis_name="c", subcore_axis_name="s")
    run = pl.kernel(
        _sc_body,
        out_type=jax.ShapeDtypeStruct((BATCH, D_MODEL), jnp.float32),
        mesh=mesh,
        scratch_types=[
            pltpu.VMEM((CHUNKS_PER_WORKER, CHUNK), jnp.int32),
            pltpu.VMEM((CHUNKS_PER_WORKER, CHUNK), jnp.int32),
            pltpu.VMEM((2, CHUNK, D_MODEL), jnp.float32),
            pltpu.VMEM((2, CHUNK, HALF), jnp.float32),
            pltpu.VMEM((2, CHUNK, HALF), jnp.float32),
        ] + [pltpu.SemaphoreType.DMA] * 8,
    )
    return run(x, widx, hidx, tab)


def kernel(x, coords, pos_table):
    return _pos_encode_add(x, coords, pos_table)


# speedup vs baseline: 2.6710x; 1.0006x over previous
import jax
import jax.numpy as jnp
from jax import lax
from jax.experimental import pallas as pl
from jax.experimental.pallas import tpu as pltpu, tpu_sc as plsc

D_MODEL = 256
HALF = 128
TABLE_ROWS = 512
BATCH = 16384

_info = plsc.get_sparse_core_info()
NUM_CORES = _info.num_cores
NUM_SUBCORES = _info.num_subcores
NUM_WORKERS = NUM_CORES * NUM_SUBCORES
ROWS_PER_WORKER = BATCH // NUM_WORKERS
CHUNK = 64
CHUNKS_PER_WORKER = ROWS_PER_WORKER // CHUNK
LANES = 16


NBUF = 3


def _sc_body(x_hbm, widx_hbm, hidx_hbm, tab_hbm, out_hbm,
             widx_v, hidx_v, xb, wr, hr, *sems):
    wid = lax.axis_index("s") * NUM_CORES + lax.axis_index("c")
    base = wid * ROWS_PER_WORKER
    irow = wid * CHUNKS_PER_WORKER

    pltpu.sync_copy(widx_hbm.at[pl.ds(irow, CHUNKS_PER_WORKER)], widx_v)
    pltpu.sync_copy(hidx_hbm.at[pl.ds(irow, CHUNKS_PER_WORKER)], hidx_v)

    sx = sems[0:NBUF]
    sw = sems[NBUF:2 * NBUF]
    sh = sems[2 * NBUF:3 * NBUF]
    so = sems[3 * NBUF:4 * NBUF]

    def issue(j):
        b = j % NBUF
        return (
            pltpu.async_copy(
                x_hbm.at[pl.ds(base + j * CHUNK, CHUNK)], xb.at[b], sx[b]),
            pltpu.async_copy(tab_hbm.at[widx_v.at[j]], wr.at[b], sw[b]),
            pltpu.async_copy(tab_hbm.at[hidx_v.at[j]], hr.at[b], sh[b]),
        )

    descs = [None] * CHUNKS_PER_WORKER
    outd = [None] * NBUF
    descs[0] = issue(0)
    descs[1] = issue(1)
    for j in range(CHUNKS_PER_WORKER):
        b = j % NBUF
        if j + 2 < CHUNKS_PER_WORKER:
            nb = (j + 2) % NBUF
            if outd[nb] is not None:
                outd[nb].wait()
            descs[j + 2] = issue(j + 2)
        for d in descs[j]:
            d.wait()

        @plsc.parallel_loop(0, CHUNK, unroll=8)
        def row_body(r):
            for t in range(HALF // LANES):
                o = t * LANES
                xb[b, r, pl.ds(o, LANES)] = (
                    xb[b, r, pl.ds(o, LANES)] + wr[b, r, pl.ds(o, LANES)])
                xb[b, r, pl.ds(HALF + o, LANES)] = (
                    xb[b, r, pl.ds(HALF + o, LANES)]
                    + hr[b, r, pl.ds(o, LANES)])

        outd[b] = pltpu.async_copy(
            xb.at[b], out_hbm.at[pl.ds(base + j * CHUNK, CHUNK)], so[b])
    for d in outd:
        if d is not None:
            d.wait()


@jax.jit
def _pos_encode_add(x, coords, pos_table):
    tab = jnp.transpose(jnp.concatenate(
        [pos_table[:HALF, 0, :], pos_table[HALF:, :, 0]], axis=1))
    widx = coords[:, 3].reshape(BATCH // CHUNK, CHUNK)
    hidx = (coords[:, 2] + TABLE_ROWS).reshape(BATCH // CHUNK, CHUNK)

    mesh = plsc.VectorSubcoreMesh(core_axis_name="c", subcore_axis_name="s")
    run = pl.kernel(
        _sc_body,
        out_type=jax.ShapeDtypeStruct((BATCH, D_MODEL), jnp.float32),
        mesh=mesh,
        scratch_types=[
            pltpu.VMEM((CHUNKS_PER_WORKER, CHUNK), jnp.int32),
            pltpu.VMEM((CHUNKS_PER_WORKER, CHUNK), jnp.int32),
            pltpu.VMEM((NBUF, CHUNK, D_MODEL), jnp.float32),
            pltpu.VMEM((NBUF, CHUNK, HALF), jnp.float32),
            pltpu.VMEM((NBUF, CHUNK, HALF), jnp.float32),
        ] + [pltpu.SemaphoreType.DMA] * (4 * NBUF),
    )
    return run(x, widx, hidx, tab)


def kernel(x, coords, pos_table):
    return _pos_encode_add(x, coords, pos_table)
